# Initial kernel scaffold; baseline (speedup 1.0000x reference)
import jax
import jax.numpy as jnp
from jax.experimental import pallas as pl


def kernel(x, gate_W, gate_b, W1, b1, W2, b2, W3, b3):
    raise NotImplementedError("write your pallas kernel here")


# speedup vs baseline: 1.2747x; 1.2747x over previous
import jax
import jax.numpy as jnp
import numpy as np
from jax.experimental import pallas as pl
from jax.experimental.pallas import tpu as pltpu

_N_EXPERTS = 8
_N_ACTIVE = 5
_TEMP = float(np.e)
_TB = 512


def _moe_body(x_ref, gw_ref, gb_ref, w1_ref, b1_ref, w2_ref, b2_ref,
              w3_ref, b3_ref, out_ref, wts_ref):
    e = pl.program_id(1)

    @pl.when(e == 0)
    def _gate():
        scores = (jnp.dot(x_ref[...], gw_ref[...],
                          preferred_element_type=jnp.float32)
                  + gb_ref[...]) / _TEMP
        m = jnp.max(scores, axis=-1, keepdims=True)
        ex = jnp.exp(scores - m)
        probs = ex / jnp.sum(ex, axis=-1, keepdims=True)
        idx = jax.lax.broadcasted_iota(jnp.int32, probs.shape, 1)
        cols = []
        for i in range(_N_EXPERTS):
            pi = probs[:, i:i + 1]
            beats = (probs > pi).astype(jnp.float32) + \
                jnp.where(probs == pi, (idx < i).astype(jnp.float32), 0.0)
            rank = jnp.sum(beats, axis=-1, keepdims=True)
            cols.append((rank < _N_ACTIVE).astype(jnp.float32))
        mask = jnp.concatenate(cols, axis=-1)
        w = probs * mask
        wts_ref[...] = w / (jnp.sum(w, axis=-1, keepdims=True) + 1e-8)

    xb = x_ref[...]
    h1 = jnp.maximum(
        jnp.dot(xb, w1_ref[0], preferred_element_type=jnp.float32)
        + b1_ref[0], 0.0)
    h2 = jnp.maximum(
        jnp.dot(h1, w2_ref[0], preferred_element_type=jnp.float32)
        + b2_ref[0], 0.0)
    o = jnp.dot(h2, w3_ref[0], preferred_element_type=jnp.float32) \
        + b3_ref[0]

    onehot = (jax.lax.broadcasted_iota(jnp.int32, (_TB, _N_EXPERTS), 1)
              == e).astype(jnp.float32)
    w_col = jnp.sum(wts_ref[...] * onehot, axis=-1, keepdims=True)
    contrib = w_col * o

    @pl.when(e == 0)
    def _init():
        out_ref[...] = contrib

    @pl.when(e != 0)
    def _acc():
        out_ref[...] += contrib


def kernel(x, gate_W, gate_b, W1, b1, W2, b2, W3, b3):
    n, d = x.shape
    grid = (n // _TB, _N_EXPERTS)
    return pl.pallas_call(
        _moe_body,
        grid=grid,
        in_specs=[
            pl.BlockSpec((_TB, d), lambda t, e: (t, 0)),
            pl.BlockSpec((d, _N_EXPERTS), lambda t, e: (0, 0)),
            pl.BlockSpec((1, _N_EXPERTS), lambda t, e: (0, 0)),
            pl.BlockSpec((1, d, W1.shape[2]), lambda t, e: (e, 0, 0)),
            pl.BlockSpec((1, 1, b1.shape[1]), lambda t, e: (e, 0, 0)),
            pl.BlockSpec((1, W2.shape[1], W2.shape[2]), lambda t, e: (e, 0, 0)),
            pl.BlockSpec((1, 1, b2.shape[1]), lambda t, e: (e, 0, 0)),
            pl.BlockSpec((1, W3.shape[1], W3.shape[2]), lambda t, e: (e, 0, 0)),
            pl.BlockSpec((1, 1, b3.shape[1]), lambda t, e: (e, 0, 0)),
        ],
        out_specs=pl.BlockSpec((_TB, W3.shape[2]), lambda t, e: (t, 0)),
        out_shape=jax.ShapeDtypeStruct((n, W3.shape[2]), jnp.float32),
        scratch_shapes=[pltpu.VMEM((_TB, _N_EXPERTS), jnp.float32)],
        compiler_params=pltpu.CompilerParams(
            dimension_semantics=("parallel", "arbitrary"),
            vmem_limit_bytes=100 * 1024 * 1024,
        ),
    )(x, gate_W, gate_b.reshape(1, -1), W1, b1[:, None, :], W2,
      b2[:, None, :], W3, b3[:, None, :])


# speedup vs baseline: 1.2781x; 1.0027x over previous
import jax
import jax.numpy as jnp
import numpy as np
from jax.experimental import pallas as pl
from jax.experimental.pallas import tpu as pltpu

_N_EXPERTS = 8
_N_ACTIVE = 5
_TEMP = float(np.e)
_TB = 512


def _moe_body(x_ref, gw_ref, gb_ref, w1_ref, b1_ref, w2_ref, b2_ref,
              w3_ref, b3_ref, out_ref, wts_ref):
    e = pl.program_id(1)

    @pl.when(e == 0)
    def _gate():
        scores = (jnp.dot(x_ref[...], gw_ref[...],
                          preferred_element_type=jnp.float32)
                  + gb_ref[...]) / _TEMP
        m = jnp.max(scores, axis=-1, keepdims=True)
        ex = jnp.exp(scores - m)
        probs = ex / jnp.sum(ex, axis=-1, keepdims=True)
        idx = jax.lax.broadcasted_iota(jnp.int32, probs.shape, 1)
        cols = []
        for i in range(_N_EXPERTS):
            pi = probs[:, i:i + 1]
            beats = (probs > pi).astype(jnp.float32) + \
                jnp.where(probs == pi, (idx < i).astype(jnp.float32), 0.0)
            rank = jnp.sum(beats, axis=-1, keepdims=True)
            cols.append((rank < _N_ACTIVE).astype(jnp.float32))
        mask = jnp.concatenate(cols, axis=-1)
        w = probs * mask
        wts_ref[...] = w / (jnp.sum(w, axis=-1, keepdims=True) + 1e-8)

    xb = x_ref[...].astype(jnp.bfloat16)
    h1 = jnp.maximum(
        jnp.dot(xb, w1_ref[0].astype(jnp.bfloat16),
                preferred_element_type=jnp.float32)
        + b1_ref[0], 0.0).astype(jnp.bfloat16)
    h2 = jnp.maximum(
        jnp.dot(h1, w2_ref[0].astype(jnp.bfloat16),
                preferred_element_type=jnp.float32)
        + b2_ref[0], 0.0).astype(jnp.bfloat16)
    o = jnp.dot(h2, w3_ref[0].astype(jnp.bfloat16),
                preferred_element_type=jnp.float32) + b3_ref[0]

    onehot = (jax.lax.broadcasted_iota(jnp.int32, (_TB, _N_EXPERTS), 1)
              == e).astype(jnp.float32)
    w_col = jnp.sum(wts_ref[...] * onehot, axis=-1, keepdims=True)
    contrib = w_col * o

    @pl.when(e == 0)
    def _init():
        out_ref[...] = contrib

    @pl.when(e != 0)
    def _acc():
        out_ref[...] += contrib


def kernel(x, gate_W, gate_b, W1, b1, W2, b2, W3, b3):
    n, d = x.shape
    grid = (n // _TB, _N_EXPERTS)
    return pl.pallas_call(
        _moe_body,
        grid=grid,
        in_specs=[
            pl.BlockSpec((_TB, d), lambda t, e: (t, 0)),
            pl.BlockSpec((d, _N_EXPERTS), lambda t, e: (0, 0)),
            pl.BlockSpec((1, _N_EXPERTS), lambda t, e: (0, 0)),
            pl.BlockSpec((1, d, W1.shape[2]), lambda t, e: (e, 0, 0)),
            pl.BlockSpec((1, 1, b1.shape[1]), lambda t, e: (e, 0, 0)),
            pl.BlockSpec((1, W2.shape[1], W2.shape[2]), lambda t, e: (e, 0, 0)),
            pl.BlockSpec((1, 1, b2.shape[1]), lambda t, e: (e, 0, 0)),
            pl.BlockSpec((1, W3.shape[1], W3.shape[2]), lambda t, e: (e, 0, 0)),
            pl.BlockSpec((1, 1, b3.shape[1]), lambda t, e: (e, 0, 0)),
        ],
        out_specs=pl.BlockSpec((_TB, W3.shape[2]), lambda t, e: (t, 0)),
        out_shape=jax.ShapeDtypeStruct((n, W3.shape[2]), jnp.float32),
        scratch_shapes=[pltpu.VMEM((_TB, _N_EXPERTS), jnp.float32)],
        compiler_params=pltpu.CompilerParams(
            dimension_semantics=("parallel", "arbitrary"),
            vmem_limit_bytes=100 * 1024 * 1024,
        ),
    )(x, gate_W, gate_b.reshape(1, -1), W1, b1[:, None, :], W2,
      b2[:, None, :], W3, b3[:, None, :])


# speedup vs baseline: 1.4633x; 1.1448x over previous
import jax
import jax.numpy as jnp
import numpy as np
from jax.experimental import pallas as pl
from jax.experimental.pallas import tpu as pltpu

_N_EXPERTS = 8
_N_ACTIVE = 5
_TEMP = float(np.e)
_TB = 1024


def _moe_body(x_ref, gw_ref, gb_ref, w1_ref, b1_ref, w2_ref, b2_ref,
              w3_ref, b3_ref, out_ref, wts_ref):
    e = pl.program_id(1)

    @pl.when(e == 0)
    def _gate():
        scores = (jnp.dot(x_ref[...], gw_ref[...],
                          preferred_element_type=jnp.float32)
                  + gb_ref[...]) / _TEMP
        m = jnp.max(scores, axis=-1, keepdims=True)
        ex = jnp.exp(scores - m)
        probs = ex / jnp.sum(ex, axis=-1, keepdims=True)
        idx = jax.lax.broadcasted_iota(jnp.int32, probs.shape, 1)
        cols = []
        for i in range(_N_EXPERTS):
            pi = probs[:, i:i + 1]
            beats = (probs > pi).astype(jnp.float32) + \
                jnp.where(probs == pi, (idx < i).astype(jnp.float32), 0.0)
            rank = jnp.sum(beats, axis=-1, keepdims=True)
            cols.append((rank < _N_ACTIVE).astype(jnp.float32))
        mask = jnp.concatenate(cols, axis=-1)
        w = probs * mask
        wts_ref[...] = w / (jnp.sum(w, axis=-1, keepdims=True) + 1e-8)

    xb = x_ref[...].astype(jnp.bfloat16)
    h1 = jnp.maximum(
        jnp.dot(xb, w1_ref[0].astype(jnp.bfloat16),
                preferred_element_type=jnp.float32)
        + b1_ref[0], 0.0).astype(jnp.bfloat16)
    h2 = jnp.maximum(
        jnp.dot(h1, w2_ref[0].astype(jnp.bfloat16),
                preferred_element_type=jnp.float32)
        + b2_ref[0], 0.0).astype(jnp.bfloat16)
    o = jnp.dot(h2, w3_ref[0].astype(jnp.bfloat16),
                preferred_element_type=jnp.float32) + b3_ref[0]

    onehot = (jax.lax.broadcasted_iota(jnp.int32, (_TB, _N_EXPERTS), 1)
              == e).astype(jnp.float32)
    w_col = jnp.sum(wts_ref[...] * onehot, axis=-1, keepdims=True)
    contrib = w_col * o

    @pl.when(e == 0)
    def _init():
        out_ref[...] = contrib

    @pl.when(e != 0)
    def _acc():
        out_ref[...] += contrib


def kernel(x, gate_W, gate_b, W1, b1, W2, b2, W3, b3):
    n, d = x.shape
    grid = (n // _TB, _N_EXPERTS)
    return pl.pallas_call(
        _moe_body,
        grid=grid,
        in_specs=[
            pl.BlockSpec((_TB, d), lambda t, e: (t, 0)),
            pl.BlockSpec((d, _N_EXPERTS), lambda t, e: (0, 0)),
            pl.BlockSpec((1, _N_EXPERTS), lambda t, e: (0, 0)),
            pl.BlockSpec((1, d, W1.shape[2]), lambda t, e: (e, 0, 0)),
            pl.BlockSpec((1, 1, b1.shape[1]), lambda t, e: (e, 0, 0)),
            pl.BlockSpec((1, W2.shape[1], W2.shape[2]), lambda t, e: (e, 0, 0)),
            pl.BlockSpec((1, 1, b2.shape[1]), lambda t, e: (e, 0, 0)),
            pl.BlockSpec((1, W3.shape[1], W3.shape[2]), lambda t, e: (e, 0, 0)),
            pl.BlockSpec((1, 1, b3.shape[1]), lambda t, e: (e, 0, 0)),
        ],
        out_specs=pl.BlockSpec((_TB, W3.shape[2]), lambda t, e: (t, 0)),
        out_shape=jax.ShapeDtypeStruct((n, W3.shape[2]), jnp.float32),
        scratch_shapes=[pltpu.VMEM((_TB, _N_EXPERTS), jnp.float32)],
        compiler_params=pltpu.CompilerParams(
            dimension_semantics=("parallel", "arbitrary"),
            vmem_limit_bytes=100 * 1024 * 1024,
        ),
    )(x, gate_W, gate_b.reshape(1, -1), W1, b1[:, None, :], W2,
      b2[:, None, :], W3, b3[:, None, :])
